# baseline (speedup 1.0000x reference)
import functools

import jax
import jax.numpy as jnp
from jax import lax
from jax.experimental import pallas as pl
from jax.experimental.pallas import tpu as pltpu
from jax.experimental.pallas import tpu_sc as plsc

N = 10000
E = 160000
LANES = 128

NC = 2
NS = 16
VL = 16
NP = 10240
BE = 64
NBLK = 80
EP = NC * NS * NBLK * BE
EDEG = EP // NS
ENRM = EP // (NC * NS)
RB = NP // NS
ZR = 32
NH = NP // 2


def _layer_body(a_ref, h_ref, sw_ref, wc_ref, wr_ref, b_ref, o_ref, acc_ref):
    k = pl.program_id(2)
    nk = pl.num_programs(2)

    @pl.when(k == 0)
    def _():
        acc_ref[...] = jnp.zeros_like(acc_ref)

    h = h_ref[0]
    hagg = a_ref[0, 0] + a_ref[0, 1] + sw_ref[...] * h
    acc_ref[...] += jnp.dot(hagg, wc_ref[...], preferred_element_type=jnp.float32)
    acc_ref[...] += jnp.dot(h, wr_ref[...], preferred_element_type=jnp.float32)

    @pl.when(k == nk - 1)
    def _():
        r = jnp.maximum(acc_ref[...] + b_ref[...], 0.0)
        for c in range(o_ref.shape[0]):
            o_ref[c] = r[:, c * LANES:(c + 1) * LANES]


def _fused_layer(A, H, selfw, Wc, Wr, b, *, bn=2000, bco=1024):
    C = A.shape[0]
    fout = Wc.shape[1]
    co = fout // bco
    grid = (N // bn, co, C)
    out = pl.pallas_call(
        _layer_body,
        grid=grid,
        in_specs=[
            pl.BlockSpec((1, 2, bn, LANES), lambda n, c, k: (k, 0, n, 0)),
            pl.BlockSpec((1, bn, LANES), lambda n, c, k: (k, n, 0)),
            pl.BlockSpec((bn, 1), lambda n, c, k: (n, 0)),
            pl.BlockSpec((LANES, bco), lambda n, c, k: (k, c)),
            pl.BlockSpec((LANES, bco), lambda n, c, k: (k, c)),
            pl.BlockSpec((1, bco), lambda n, c, k: (0, c)),
        ],
        out_specs=pl.BlockSpec(
            (bco // LANES, bn, LANES), lambda n, c, k: (c, n, 0)
        ),
        out_shape=jax.ShapeDtypeStruct((fout // LANES, N, LANES), jnp.float32),
        scratch_shapes=[pltpu.VMEM((bn, bco), jnp.float32)],
        compiler_params=pltpu.CompilerParams(
            dimension_semantics=("parallel", "parallel", "arbitrary"),
        ),
    )(A, H, selfw.reshape(N, 1), Wc, Wr, b.reshape(1, fout))
    return out


def _mm_body(h_ref, w_ref, o_ref, acc_ref):
    k = pl.program_id(2)
    nk = pl.num_programs(2)

    @pl.when(k == 0)
    def _():
        acc_ref[...] = jnp.zeros_like(acc_ref)

    acc_ref[...] += jnp.dot(h_ref[0], w_ref[...], preferred_element_type=jnp.float32)

    @pl.when(k == nk - 1)
    def _():
        r = acc_ref[...]
        for c in range(o_ref.shape[0]):
            o_ref[c] = r[:, c * LANES:(c + 1) * LANES]


def _mm_chunked(H, W, *, bn=2000, bco=1024):
    C = H.shape[0]
    fout = W.shape[1]
    grid = (N // bn, fout // bco, C)
    return pl.pallas_call(
        _mm_body,
        grid=grid,
        in_specs=[
            pl.BlockSpec((1, bn, LANES), lambda n, c, k: (k, n, 0)),
            pl.BlockSpec((LANES, bco), lambda n, c, k: (k, c)),
        ],
        out_specs=pl.BlockSpec((bco // LANES, bn, LANES), lambda n, c, k: (c, n, 0)),
        out_shape=jax.ShapeDtypeStruct((fout // LANES, N, LANES), jnp.float32),
        scratch_shapes=[pltpu.VMEM((bn, bco), jnp.float32)],
        compiler_params=pltpu.CompilerParams(
            dimension_semantics=("parallel", "parallel", "arbitrary"),
        ),
    )(H, W)


def _l4_body(h_ref, wr_ref, a_ref, xt_ref, sw_ref, b_ref, o_ref, acc_ref):
    k = pl.program_id(2)
    nk = pl.num_programs(2)

    @pl.when(k == 0)
    def _():
        acc_ref[...] = jnp.zeros_like(acc_ref)

    acc_ref[...] += jnp.dot(h_ref[0], wr_ref[...], preferred_element_type=jnp.float32)

    @pl.when(k == nk - 1)
    def _():
        sw = sw_ref[...]
        r = acc_ref[...] + b_ref[...]
        for c in range(o_ref.shape[0]):
            extra = a_ref[c, 0] + a_ref[c, 1] + sw * xt_ref[c]
            o_ref[c] = jnp.maximum(r[:, c * LANES:(c + 1) * LANES] + extra, 0.0)


def _l4_combine(H3, Wr4, A4, XT, selfw, b, *, bn=2000, bco=512):
    C = H3.shape[0]
    fout = Wr4.shape[1]
    grid = (N // bn, fout // bco, C)
    return pl.pallas_call(
        _l4_body,
        grid=grid,
        in_specs=[
            pl.BlockSpec((1, bn, LANES), lambda n, c, k: (k, n, 0)),
            pl.BlockSpec((LANES, bco), lambda n, c, k: (k, c)),
            pl.BlockSpec((bco // LANES, 2, bn, LANES), lambda n, c, k: (c, 0, n, 0)),
            pl.BlockSpec((bco // LANES, bn, LANES), lambda n, c, k: (c, n, 0)),
            pl.BlockSpec((bn, 1), lambda n, c, k: (n, 0)),
            pl.BlockSpec((1, bco), lambda n, c, k: (0, c)),
        ],
        out_specs=pl.BlockSpec((bco // LANES, bn, LANES), lambda n, c, k: (c, n, 0)),
        out_shape=jax.ShapeDtypeStruct((fout // LANES, N, LANES), jnp.float32),
        scratch_shapes=[pltpu.VMEM((bn, bco), jnp.float32)],
        compiler_params=pltpu.CompilerParams(
            dimension_semantics=("parallel", "parallel", "arbitrary"),
        ),
    )(H3, Wr4, A4, XT, selfw.reshape(N, 1), b.reshape(1, fout))


def _proj_body(h_ref, w_ref, b_ref, o_ref, acc_ref):
    k = pl.program_id(1)
    nk = pl.num_programs(1)

    @pl.when(k == 0)
    def _():
        acc_ref[...] = jnp.zeros_like(acc_ref)

    acc_ref[...] += jnp.dot(h_ref[0], w_ref[...], preferred_element_type=jnp.float32)

    @pl.when(k == nk - 1)
    def _():
        o_ref[...] = acc_ref[...] + b_ref[...]


def _proj(H, Wo, bo, *, bn=2000):
    C = H.shape[0]
    fout = Wo.shape[1]
    grid = (N // bn, C)
    return pl.pallas_call(
        _proj_body,
        grid=grid,
        in_specs=[
            pl.BlockSpec((1, bn, LANES), lambda n, k: (k, n, 0)),
            pl.BlockSpec((LANES, fout), lambda n, k: (k, 0)),
            pl.BlockSpec((1, fout), lambda n, k: (0, 0)),
        ],
        out_specs=pl.BlockSpec((bn, fout), lambda n, k: (n, 0)),
        out_shape=jax.ShapeDtypeStruct((N, fout), jnp.float32),
        scratch_shapes=[pltpu.VMEM((bn, fout), jnp.float32)],
        compiler_params=pltpu.CompilerParams(
            dimension_semantics=("parallel", "arbitrary"),
        ),
    )(H, Wo, bo.reshape(1, fout))


_MESH = dict(core_axis_name="c", subcore_axis_name="s")


def _rsqrt16(x):
    i = plsc.bitcast(x, jnp.int32)
    y = plsc.bitcast(jnp.int32(0x5F3759DF) - (i >> 1), jnp.float32)
    for _ in range(3):
        y = y * (1.5 - 0.5 * x * y * y)
    return y


def _sc_norm(src, dst, w):
    def body(src_hbm, dst_hbm, w_hbm, norm_hbm, selfw_hbm,
             dstd_v, wd_v, deg_v, red_v, diss_v, sw_v, dis_v,
             srcn_v, dstn_v, wn_v, nout_v, shared_deg, shared_dis):
        ci = lax.axis_index("c")
        si = lax.axis_index("s")
        wid = ci * NS + si

        pltpu.sync_copy(dst_hbm.at[pl.ds(si * EDEG, EDEG)], dstd_v)
        pltpu.sync_copy(w_hbm.at[pl.ds(si * EDEG, EDEG)], wd_v)

        @pl.loop(0, NP // VL)
        def _(j):
            deg_v[pl.ds(j * VL, VL)] = jnp.zeros((VL,), jnp.float32)

        @pl.loop(0, EDEG // VL)
        def _(j):
            dv = dstd_v[pl.ds(j * VL, VL)]
            wv = wd_v[pl.ds(j * VL, VL)]
            plsc.addupdate_scatter(deg_v, [dv], wv)

        pltpu.sync_copy(deg_v, shared_deg.at[si])
        plsc.subcore_barrier()

        for t in range(NS):
            pltpu.sync_copy(shared_deg.at[t, pl.ds(si * RB, RB)], red_v.at[t])

        @pl.loop(0, RB // VL)
        def _(j):
            sl = pl.ds(j * VL, VL)
            acc = red_v[0, sl]
            for t in range(1, NS):
                acc = acc + red_v[t, sl]
            degt = acc + 1.0
            y = _rsqrt16(degt)
            diss_v[sl] = y
            sw_v[sl] = y * y

        half = RB // NC
        pltpu.sync_copy(sw_v.at[pl.ds(ci * half, half)],
                        selfw_hbm.at[pl.ds(si * RB + ci * half, half)])
        pltpu.sync_copy(diss_v, shared_dis.at[pl.ds(si * RB, RB)])
        plsc.subcore_barrier()
        pltpu.sync_copy(shared_dis, dis_v)

        pltpu.sync_copy(src_hbm.at[pl.ds(wid * ENRM, ENRM)], srcn_v)
        pltpu.sync_copy(dst_hbm.at[pl.ds(wid * ENRM, ENRM)], dstn_v)
        pltpu.sync_copy(w_hbm.at[pl.ds(wid * ENRM, ENRM)], wn_v)

        @pl.loop(0, ENRM // VL)
        def _(j):
            sl = pl.ds(j * VL, VL)
            sv = srcn_v[sl]
            dv = dstn_v[sl]
            wv = wn_v[sl]
            nout_v[sl] = (plsc.load_gather(dis_v, [sv]) * wv
                          * plsc.load_gather(dis_v, [dv]))

        pltpu.sync_copy(nout_v, norm_hbm.at[pl.ds(wid * ENRM, ENRM)])

    f = pl.kernel(
        body,
        out_type=(jax.ShapeDtypeStruct((EP,), jnp.float32),
                  jax.ShapeDtypeStruct((NP,), jnp.float32)),
        mesh=plsc.VectorSubcoreMesh(**_MESH),
        compiler_params=pltpu.CompilerParams(needs_layout_passes=False),
        scratch_types=[
            pltpu.VMEM((EDEG,), jnp.int32),
            pltpu.VMEM((EDEG,), jnp.float32),
            pltpu.VMEM((NP,), jnp.float32),
            pltpu.VMEM((NS, RB), jnp.float32),
            pltpu.VMEM((RB,), jnp.float32),
            pltpu.VMEM((RB,), jnp.float32),
            pltpu.VMEM((NP,), jnp.float32),
            pltpu.VMEM((ENRM,), jnp.int32),
            pltpu.VMEM((ENRM,), jnp.int32),
            pltpu.VMEM((ENRM,), jnp.float32),
            pltpu.VMEM((ENRM,), jnp.float32),
            pltpu.VMEM_SHARED((NS, NP), jnp.float32),
            pltpu.VMEM_SHARED((NP,), jnp.float32),
        ],
    )
    return f(src, dst, w)


def _sc_agg(Hc, srcb, dstb, normb):
    C = Hc.shape[0]
    hflat = Hc.reshape(C * N, LANES)

    def body(h_hbm, srcb_hbm, dstb_hbm, normb_hbm, out_hbm,
             src_adj, dst_loc, dst_adj, norm_loc, gbuf, zbuf, acc,
             g0, g1, g2, g3, s0, s1, s2, s3):
        gs = (g0, g1, g2, g3)
        ss = (s0, s1, s2, s3)
        ci = lax.axis_index("c")
        si = lax.axis_index("s")
        wid = ci * NS + si
        srows = NH // NS
        base = si * srows

        pltpu.sync_copy(srcb_hbm.at[wid], src_adj)
        pltpu.sync_copy(dstb_hbm.at[wid], dst_loc)
        pltpu.sync_copy(normb_hbm.at[wid], norm_loc)

        @pl.loop(0, ZR)
        def _(j):
            for k in range(LANES // VL):
                zbuf[j, pl.ds(k * VL, VL)] = jnp.zeros((VL,), jnp.float32)

        def start_gather(jb, bi, sem):
            pltpu.async_copy(h_hbm.at[src_adj.at[jb]], gbuf.at[bi], sem)

        def wait_gather(bi, sem):
            pltpu.make_async_copy(h_hbm.at[pl.ds(0, BE)], gbuf.at[bi], sem).wait()

        def scale(jb, bi):
            @pl.loop(0, BE // VL)
            def _(g):
                nv = norm_loc[jb, pl.ds(g * VL, VL)]
                e0 = g * VL
                for lane in range(VL):
                    s = nv[lane]
                    for k in range(LANES // VL):
                        sl = pl.ds(k * VL, VL)
                        gbuf[bi, e0 + lane, sl] = gbuf[bi, e0 + lane, sl] * s

        def start_scatter(jb, bi, sem):
            pltpu.async_copy(gbuf.at[bi], acc.at[dst_adj.at[jb]], sem, add=True)

        def wait_scatter(bi, sem):
            pltpu.make_async_copy(gbuf.at[bi], acc.at[pl.ds(0, BE)], sem).wait()

        @pl.loop(0, C)
        def _(ch):
            @pl.when(ch > 0)
            def _():
                @pl.loop(0, NBLK)
                def _(j):
                    for k in range(BE // VL):
                        sl = pl.ds(k * VL, VL)
                        src_adj[j, sl] = src_adj[j, sl] + N

            for hf in range(2):
                nb = hf * NH

                @pl.loop(0, NBLK)
                def _(j):
                    for k in range(BE // VL):
                        sl = pl.ds(k * VL, VL)
                        dv = dst_loc[j, sl]
                        rel = dv - nb
                        ok = (rel >= 0) & (rel < NH)
                        dst_adj[j, sl] = jnp.where(ok, rel, NH)

                for k in range(srows // ZR):
                    pltpu.sync_copy(zbuf, acc.at[pl.ds(base + k * ZR, ZR)])

                @pl.when(si == 0)
                def _():
                    pltpu.sync_copy(zbuf.at[pl.ds(0, 8)], acc.at[pl.ds(NH, 8)])

                plsc.subcore_barrier()

                for k in range(3):
                    start_gather(k, k, gs[k])

                @pl.loop(0, NBLK, step=4)
                def _(j):
                    for k in range(4):
                        m = (k + 3) % 4
                        wait_gather(k, gs[k])
                        scale(j + k, k)
                        start_scatter(j + k, k, ss[k])

                        @pl.when(j + k + 3 < NBLK)
                        def _():
                            @pl.when(j + k >= 1)
                            def _():
                                wait_scatter(m, ss[m])

                            start_gather(j + k + 3, m, gs[m])

                for k in range(4):
                    wait_scatter(k, ss[k])

                plsc.subcore_barrier()
                for k in range(srows // ZR):
                    sl = pl.ds(base + k * ZR, ZR)
                    osl = pl.ds(nb + base + k * ZR, ZR)
                    pltpu.sync_copy(acc.at[sl], out_hbm.at[ch, ci, osl])
                plsc.subcore_barrier()

    f = pl.kernel(
        body,
        out_type=jax.ShapeDtypeStruct((C, 2, NP, LANES), jnp.float32),
        mesh=plsc.VectorSubcoreMesh(**_MESH),
        compiler_params=pltpu.CompilerParams(needs_layout_passes=False),
        scratch_types=[
            pltpu.VMEM((NBLK, BE), jnp.int32),
            pltpu.VMEM((NBLK, BE), jnp.int32),
            pltpu.VMEM((NBLK, BE), jnp.int32),
            pltpu.VMEM((NBLK, BE), jnp.float32),
            pltpu.VMEM((4, BE, LANES), jnp.float32),
            pltpu.VMEM((ZR, LANES), jnp.float32),
            pltpu.VMEM_SHARED((NH + 8, LANES), jnp.float32),
            pltpu.SemaphoreType.DMA,
            pltpu.SemaphoreType.DMA,
            pltpu.SemaphoreType.DMA,
            pltpu.SemaphoreType.DMA,
            pltpu.SemaphoreType.DMA,
            pltpu.SemaphoreType.DMA,
            pltpu.SemaphoreType.DMA,
            pltpu.SemaphoreType.DMA,
        ],
    )
    return f(hflat, srcb, dstb, normb)


def kernel(x, edge_index, edge_attr, Wc1, bc1, Wr1, br1, Wc2, bc2, Wr2, br2,
           Wc3, bc3, Wr3, br3, Wc4, bc4, Wr4, br4, Wo, bo):
    pad = EP - E
    src = jnp.concatenate([edge_index[0], jnp.zeros((pad,), jnp.int32)])
    dst = jnp.concatenate([edge_index[1], jnp.zeros((pad,), jnp.int32)])
    w = jnp.concatenate([edge_attr, jnp.zeros((pad,), jnp.float32)])

    norm, selfw_p = _sc_norm(src, dst, w)
    selfw = selfw_p[:N]
    srcb = src.reshape(NC * NS, NBLK, BE)
    dstb = dst.reshape(NC * NS, NBLK, BE)
    normb = norm.reshape(NC * NS, NBLK, BE)

    def agg(Hc):
        return _sc_agg(Hc, srcb, dstb, normb)

    H0 = x.reshape(1, N, LANES)
    A0 = agg(H0)
    H1 = _fused_layer(A0, H0, selfw, Wc1, Wr1, bc1 + br1)

    A1 = agg(H1)
    H2 = _fused_layer(A1, H1, selfw, Wc2, Wr2, bc2 + br2)

    A2 = agg(H2)
    H3 = _fused_layer(A2, H2, selfw, Wc3, Wr3, bc3 + br3)

    XT = _mm_chunked(H3, Wc4)
    A4 = agg(XT)
    H4 = _l4_combine(H3, Wr4, A4, XT, selfw, bc4 + br4)

    return _proj(H4, Wo, bo)


# speedup vs baseline: 1.0101x; 1.0101x over previous
import functools

import jax
import jax.numpy as jnp
from jax import lax
from jax.experimental import pallas as pl
from jax.experimental.pallas import tpu as pltpu
from jax.experimental.pallas import tpu_sc as plsc

N = 10000
E = 160000
LANES = 128

NC = 2
NS = 16
VL = 16
NP = 10240
BE = 64
NBLK = 80
EP = NC * NS * NBLK * BE
EDEG = EP // NS
ENRM = EP // (NC * NS)
RB = NP // NS
ZR = 64
NH = NP // 2


def _layer_body(a_ref, h_ref, sw_ref, wc_ref, wr_ref, b_ref, o_ref, acc_ref):
    k = pl.program_id(2)
    nk = pl.num_programs(2)

    @pl.when(k == 0)
    def _():
        acc_ref[...] = jnp.zeros_like(acc_ref)

    h = h_ref[0]
    hagg = a_ref[0, 0] + a_ref[0, 1] + sw_ref[...] * h
    acc_ref[...] += jnp.dot(hagg, wc_ref[...], preferred_element_type=jnp.float32)
    acc_ref[...] += jnp.dot(h, wr_ref[...], preferred_element_type=jnp.float32)

    @pl.when(k == nk - 1)
    def _():
        r = jnp.maximum(acc_ref[...] + b_ref[...], 0.0)
        for c in range(o_ref.shape[0]):
            o_ref[c] = r[:, c * LANES:(c + 1) * LANES]


def _fused_layer(A, H, selfw, Wc, Wr, b, *, bn=2000, bco=1024):
    C = A.shape[0]
    fout = Wc.shape[1]
    co = fout // bco
    grid = (N // bn, co, C)
    out = pl.pallas_call(
        _layer_body,
        grid=grid,
        in_specs=[
            pl.BlockSpec((1, 2, bn, LANES), lambda n, c, k: (k, 0, n, 0)),
            pl.BlockSpec((1, bn, LANES), lambda n, c, k: (k, n, 0)),
            pl.BlockSpec((bn, 1), lambda n, c, k: (n, 0)),
            pl.BlockSpec((LANES, bco), lambda n, c, k: (k, c)),
            pl.BlockSpec((LANES, bco), lambda n, c, k: (k, c)),
            pl.BlockSpec((1, bco), lambda n, c, k: (0, c)),
        ],
        out_specs=pl.BlockSpec(
            (bco // LANES, bn, LANES), lambda n, c, k: (c, n, 0)
        ),
        out_shape=jax.ShapeDtypeStruct((fout // LANES, N, LANES), jnp.float32),
        scratch_shapes=[pltpu.VMEM((bn, bco), jnp.float32)],
        compiler_params=pltpu.CompilerParams(
            dimension_semantics=("parallel", "parallel", "arbitrary"),
        ),
    )(A, H, selfw.reshape(N, 1), Wc, Wr, b.reshape(1, fout))
    return out


def _mm_body(h_ref, w_ref, o_ref, acc_ref):
    k = pl.program_id(2)
    nk = pl.num_programs(2)

    @pl.when(k == 0)
    def _():
        acc_ref[...] = jnp.zeros_like(acc_ref)

    acc_ref[...] += jnp.dot(h_ref[0], w_ref[...], preferred_element_type=jnp.float32)

    @pl.when(k == nk - 1)
    def _():
        r = acc_ref[...]
        for c in range(o_ref.shape[0]):
            o_ref[c] = r[:, c * LANES:(c + 1) * LANES]


def _mm_chunked(H, W, *, bn=2000, bco=1024):
    C = H.shape[0]
    fout = W.shape[1]
    grid = (N // bn, fout // bco, C)
    return pl.pallas_call(
        _mm_body,
        grid=grid,
        in_specs=[
            pl.BlockSpec((1, bn, LANES), lambda n, c, k: (k, n, 0)),
            pl.BlockSpec((LANES, bco), lambda n, c, k: (k, c)),
        ],
        out_specs=pl.BlockSpec((bco // LANES, bn, LANES), lambda n, c, k: (c, n, 0)),
        out_shape=jax.ShapeDtypeStruct((fout // LANES, N, LANES), jnp.float32),
        scratch_shapes=[pltpu.VMEM((bn, bco), jnp.float32)],
        compiler_params=pltpu.CompilerParams(
            dimension_semantics=("parallel", "parallel", "arbitrary"),
        ),
    )(H, W)


def _l4_body(h_ref, wr_ref, a_ref, xt_ref, sw_ref, b_ref, o_ref, acc_ref):
    k = pl.program_id(2)
    nk = pl.num_programs(2)

    @pl.when(k == 0)
    def _():
        acc_ref[...] = jnp.zeros_like(acc_ref)

    acc_ref[...] += jnp.dot(h_ref[0], wr_ref[...], preferred_element_type=jnp.float32)

    @pl.when(k == nk - 1)
    def _():
        sw = sw_ref[...]
        r = acc_ref[...] + b_ref[...]
        for c in range(o_ref.shape[0]):
            extra = a_ref[c, 0] + a_ref[c, 1] + sw * xt_ref[c]
            o_ref[c] = jnp.maximum(r[:, c * LANES:(c + 1) * LANES] + extra, 0.0)


def _l4_combine(H3, Wr4, A4, XT, selfw, b, *, bn=2000, bco=512):
    C = H3.shape[0]
    fout = Wr4.shape[1]
    grid = (N // bn, fout // bco, C)
    return pl.pallas_call(
        _l4_body,
        grid=grid,
        in_specs=[
            pl.BlockSpec((1, bn, LANES), lambda n, c, k: (k, n, 0)),
            pl.BlockSpec((LANES, bco), lambda n, c, k: (k, c)),
            pl.BlockSpec((bco // LANES, 2, bn, LANES), lambda n, c, k: (c, 0, n, 0)),
            pl.BlockSpec((bco // LANES, bn, LANES), lambda n, c, k: (c, n, 0)),
            pl.BlockSpec((bn, 1), lambda n, c, k: (n, 0)),
            pl.BlockSpec((1, bco), lambda n, c, k: (0, c)),
        ],
        out_specs=pl.BlockSpec((bco // LANES, bn, LANES), lambda n, c, k: (c, n, 0)),
        out_shape=jax.ShapeDtypeStruct((fout // LANES, N, LANES), jnp.float32),
        scratch_shapes=[pltpu.VMEM((bn, bco), jnp.float32)],
        compiler_params=pltpu.CompilerParams(
            dimension_semantics=("parallel", "parallel", "arbitrary"),
        ),
    )(H3, Wr4, A4, XT, selfw.reshape(N, 1), b.reshape(1, fout))


def _proj_body(h_ref, w_ref, b_ref, o_ref, acc_ref):
    k = pl.program_id(1)
    nk = pl.num_programs(1)

    @pl.when(k == 0)
    def _():
        acc_ref[...] = jnp.zeros_like(acc_ref)

    acc_ref[...] += jnp.dot(h_ref[0], w_ref[...], preferred_element_type=jnp.float32)

    @pl.when(k == nk - 1)
    def _():
        o_ref[...] = acc_ref[...] + b_ref[...]


def _proj(H, Wo, bo, *, bn=2000):
    C = H.shape[0]
    fout = Wo.shape[1]
    grid = (N // bn, C)
    return pl.pallas_call(
        _proj_body,
        grid=grid,
        in_specs=[
            pl.BlockSpec((1, bn, LANES), lambda n, k: (k, n, 0)),
            pl.BlockSpec((LANES, fout), lambda n, k: (k, 0)),
            pl.BlockSpec((1, fout), lambda n, k: (0, 0)),
        ],
        out_specs=pl.BlockSpec((bn, fout), lambda n, k: (n, 0)),
        out_shape=jax.ShapeDtypeStruct((N, fout), jnp.float32),
        scratch_shapes=[pltpu.VMEM((bn, fout), jnp.float32)],
        compiler_params=pltpu.CompilerParams(
            dimension_semantics=("parallel", "arbitrary"),
        ),
    )(H, Wo, bo.reshape(1, fout))


_MESH = dict(core_axis_name="c", subcore_axis_name="s")


def _rsqrt16(x):
    i = plsc.bitcast(x, jnp.int32)
    y = plsc.bitcast(jnp.int32(0x5F3759DF) - (i >> 1), jnp.float32)
    for _ in range(3):
        y = y * (1.5 - 0.5 * x * y * y)
    return y


def _sc_norm(src, dst, w):
    def body(src_hbm, dst_hbm, w_hbm, norm_hbm, selfw_hbm,
             dstd_v, wd_v, deg_v, red_v, diss_v, sw_v, dis_v,
             srcn_v, dstn_v, wn_v, nout_v, shared_deg, shared_dis):
        ci = lax.axis_index("c")
        si = lax.axis_index("s")
        wid = ci * NS + si

        pltpu.sync_copy(dst_hbm.at[pl.ds(si * EDEG, EDEG)], dstd_v)
        pltpu.sync_copy(w_hbm.at[pl.ds(si * EDEG, EDEG)], wd_v)

        @pl.loop(0, NP // VL)
        def _(j):
            deg_v[pl.ds(j * VL, VL)] = jnp.zeros((VL,), jnp.float32)

        @pl.loop(0, EDEG // VL)
        def _(j):
            dv = dstd_v[pl.ds(j * VL, VL)]
            wv = wd_v[pl.ds(j * VL, VL)]
            plsc.addupdate_scatter(deg_v, [dv], wv)

        pltpu.sync_copy(deg_v, shared_deg.at[si])
        plsc.subcore_barrier()

        for t in range(NS):
            pltpu.sync_copy(shared_deg.at[t, pl.ds(si * RB, RB)], red_v.at[t])

        @pl.loop(0, RB // VL)
        def _(j):
            sl = pl.ds(j * VL, VL)
            acc = red_v[0, sl]
            for t in range(1, NS):
                acc = acc + red_v[t, sl]
            degt = acc + 1.0
            y = _rsqrt16(degt)
            diss_v[sl] = y
            sw_v[sl] = y * y

        half = RB // NC
        pltpu.sync_copy(sw_v.at[pl.ds(ci * half, half)],
                        selfw_hbm.at[pl.ds(si * RB + ci * half, half)])
        pltpu.sync_copy(diss_v, shared_dis.at[pl.ds(si * RB, RB)])
        plsc.subcore_barrier()
        pltpu.sync_copy(shared_dis, dis_v)

        pltpu.sync_copy(src_hbm.at[pl.ds(wid * ENRM, ENRM)], srcn_v)
        pltpu.sync_copy(dst_hbm.at[pl.ds(wid * ENRM, ENRM)], dstn_v)
        pltpu.sync_copy(w_hbm.at[pl.ds(wid * ENRM, ENRM)], wn_v)

        @pl.loop(0, ENRM // VL)
        def _(j):
            sl = pl.ds(j * VL, VL)
            sv = srcn_v[sl]
            dv = dstn_v[sl]
            wv = wn_v[sl]
            nout_v[sl] = (plsc.load_gather(dis_v, [sv]) * wv
                          * plsc.load_gather(dis_v, [dv]))

        pltpu.sync_copy(nout_v, norm_hbm.at[pl.ds(wid * ENRM, ENRM)])

    f = pl.kernel(
        body,
        out_type=(jax.ShapeDtypeStruct((EP,), jnp.float32),
                  jax.ShapeDtypeStruct((NP,), jnp.float32)),
        mesh=plsc.VectorSubcoreMesh(**_MESH),
        compiler_params=pltpu.CompilerParams(needs_layout_passes=False),
        scratch_types=[
            pltpu.VMEM((EDEG,), jnp.int32),
            pltpu.VMEM((EDEG,), jnp.float32),
            pltpu.VMEM((NP,), jnp.float32),
            pltpu.VMEM((NS, RB), jnp.float32),
            pltpu.VMEM((RB,), jnp.float32),
            pltpu.VMEM((RB,), jnp.float32),
            pltpu.VMEM((NP,), jnp.float32),
            pltpu.VMEM((ENRM,), jnp.int32),
            pltpu.VMEM((ENRM,), jnp.int32),
            pltpu.VMEM((ENRM,), jnp.float32),
            pltpu.VMEM((ENRM,), jnp.float32),
            pltpu.VMEM_SHARED((NS, NP), jnp.float32),
            pltpu.VMEM_SHARED((NP,), jnp.float32),
        ],
    )
    return f(src, dst, w)


def _sc_agg(Hc, srcb, dstb, normb):
    C = Hc.shape[0]
    hflat = Hc.reshape(C * N, LANES)

    def body(h_hbm, srcb_hbm, dstb_hbm, normb_hbm, out_hbm,
             src_loc, src_adj, dst_loc, dst_adj, norm_loc, gbuf, zbuf, acc,
             sem0, sem1):
        ci = lax.axis_index("c")
        si = lax.axis_index("s")
        wid = ci * NS + si
        srows = NH // NS
        base = si * srows

        pltpu.sync_copy(srcb_hbm.at[wid], src_loc)
        pltpu.sync_copy(dstb_hbm.at[wid], dst_loc)
        pltpu.sync_copy(normb_hbm.at[wid], norm_loc)

        @pl.loop(0, ZR)
        def _(j):
            for k in range(LANES // VL):
                zbuf[j, pl.ds(k * VL, VL)] = jnp.zeros((VL,), jnp.float32)

        def start_gather(jb, bi, sem):
            pltpu.async_copy(h_hbm.at[src_adj.at[jb]], gbuf.at[bi], sem)

        def wait_gather(bi, sem):
            pltpu.make_async_copy(h_hbm.at[pl.ds(0, BE)], gbuf.at[bi], sem).wait()

        def scale_scatter(jb, bi):
            @pl.loop(0, BE // VL)
            def _(g):
                nv = norm_loc[jb, pl.ds(g * VL, VL)]
                e0 = g * VL
                for lane in range(VL):
                    s = nv[lane]
                    for k in range(LANES // VL):
                        sl = pl.ds(k * VL, VL)
                        gbuf[bi, e0 + lane, sl] = gbuf[bi, e0 + lane, sl] * s

            pltpu.sync_copy(gbuf.at[bi], acc.at[dst_adj.at[jb]], add=True)

        @pl.loop(0, C)
        def _(ch):
            @pl.loop(0, NBLK)
            def _(j):
                for k in range(BE // VL):
                    sl = pl.ds(k * VL, VL)
                    src_adj[j, sl] = src_loc[j, sl] + ch * N

            for hf in range(2):
                nb = hf * NH

                @pl.loop(0, NBLK)
                def _(j):
                    for k in range(BE // VL):
                        sl = pl.ds(k * VL, VL)
                        dv = dst_loc[j, sl]
                        rel = dv - nb
                        ok = (rel >= 0) & (rel < NH)
                        dst_adj[j, sl] = jnp.where(ok, rel, NH)

                for k in range(srows // ZR):
                    pltpu.sync_copy(zbuf, acc.at[pl.ds(base + k * ZR, ZR)])

                @pl.when(si == 0)
                def _():
                    pltpu.sync_copy(zbuf.at[pl.ds(0, 8)], acc.at[pl.ds(NH, 8)])

                plsc.subcore_barrier()

                start_gather(0, 0, sem0)

                @pl.loop(0, NBLK, step=2)
                def _(j):
                    start_gather(j + 1, 1, sem1)
                    wait_gather(0, sem0)
                    scale_scatter(j, 0)

                    @pl.when(j + 2 < NBLK)
                    def _():
                        start_gather(j + 2, 0, sem0)

                    wait_gather(1, sem1)
                    scale_scatter(j + 1, 1)

                plsc.subcore_barrier()
                for k in range(srows // ZR):
                    sl = pl.ds(base + k * ZR, ZR)
                    osl = pl.ds(nb + base + k * ZR, ZR)
                    pltpu.sync_copy(acc.at[sl], out_hbm.at[ch, ci, osl])
                plsc.subcore_barrier()

    f = pl.kernel(
        body,
        out_type=jax.ShapeDtypeStruct((C, 2, NP, LANES), jnp.float32),
        mesh=plsc.VectorSubcoreMesh(**_MESH),
        compiler_params=pltpu.CompilerParams(needs_layout_passes=False),
        scratch_types=[
            pltpu.VMEM((NBLK, BE), jnp.int32),
            pltpu.VMEM((NBLK, BE), jnp.int32),
            pltpu.VMEM((NBLK, BE), jnp.int32),
            pltpu.VMEM((NBLK, BE), jnp.int32),
            pltpu.VMEM((NBLK, BE), jnp.float32),
            pltpu.VMEM((2, BE, LANES), jnp.float32),
            pltpu.VMEM((ZR, LANES), jnp.float32),
            pltpu.VMEM_SHARED((NH + 8, LANES), jnp.float32),
            pltpu.SemaphoreType.DMA,
            pltpu.SemaphoreType.DMA,
        ],
    )
    return f(hflat, srcb, dstb, normb)


def kernel(x, edge_index, edge_attr, Wc1, bc1, Wr1, br1, Wc2, bc2, Wr2, br2,
           Wc3, bc3, Wr3, br3, Wc4, bc4, Wr4, br4, Wo, bo):
    pad = EP - E
    src = jnp.concatenate([edge_index[0], jnp.zeros((pad,), jnp.int32)])
    dst = jnp.concatenate([edge_index[1], jnp.zeros((pad,), jnp.int32)])
    w = jnp.concatenate([edge_attr, jnp.zeros((pad,), jnp.float32)])

    norm, selfw_p = _sc_norm(src, dst, w)
    selfw = selfw_p[:N]
    srcb = src.reshape(NC * NS, NBLK, BE)
    dstb = dst.reshape(NC * NS, NBLK, BE)
    normb = norm.reshape(NC * NS, NBLK, BE)

    def agg(Hc):
        return _sc_agg(Hc, srcb, dstb, normb)

    H0 = x.reshape(1, N, LANES)
    A0 = agg(H0)
    H1 = _fused_layer(A0, H0, selfw, Wc1, Wr1, bc1 + br1)

    A1 = agg(H1)
    H2 = _fused_layer(A1, H1, selfw, Wc2, Wr2, bc2 + br2)

    A2 = agg(H2)
    H3 = _fused_layer(A2, H2, selfw, Wc3, Wr3, bc3 + br3)

    XT = _mm_chunked(H3, Wc4)
    A4 = agg(XT)
    H4 = _l4_combine(H3, Wr4, A4, XT, selfw, bc4 + br4)

    return _proj(H4, Wo, bo)
